# baseline (speedup 1.0000x reference)
import functools

import jax
import jax.numpy as jnp
from jax.experimental import pallas as pl

_BNEPS = 1e-5
_LEAK = 0.2


_PH = jax.lax.Precision.HIGHEST


def _fin(pp_ref, count):
    s = jnp.sum(pp_ref[:, 0, :], axis=0)
    ss = jnp.sum(pp_ref[:, 1, :], axis=0)
    mean = s / count
    var = ss / count - mean * mean
    den = jnp.sqrt(var + _BNEPS)
    return mean, den


def _fin2(pp_ref, count):
    B = pp_ref.shape[0]
    nb = count / B
    m = jnp.sum(pp_ref[:, 0, :], axis=0) / count
    dev = jnp.zeros_like(m)
    for b in range(B):
        s_b = pp_ref[b, 0, :]
        ss2_b = pp_ref[b, 1, :]
        c_b = pp_ref[b, 2, :]
        d = m - c_b
        dev = dev + (ss2_b - 2.0 * d * (s_b - nb * m) - nb * d * d)
    var = dev / count
    den = jnp.sqrt(var + _BNEPS)
    return m, den


def _leaky(x):
    return jnp.where(x >= 0, x, _LEAK * x)


def _rowsums(t):
    return jnp.sum(t, axis=0, keepdims=True), jnp.sum(t * t, axis=0, keepdims=True)


def _pairdist(X):
    N = X.shape[0]
    sq = jnp.sum(X * X, axis=1, keepdims=True)
    G = jax.lax.dot_general(X, X, (((1,), (1,)), ((), ())),
                            preferred_element_type=jnp.float32)
    return (sq - 2.0 * G) + jnp.reshape(sq, (1, N))


def _minsel(D, iota):
    N = D.shape[0]
    m = jnp.min(D, axis=1, keepdims=True)
    idx = jnp.min(jnp.where(D == m, iota, N), axis=1, keepdims=True)
    sel = iota == idx
    return sel, sel.astype(jnp.float32)


def _geo_body(x_ref, wsa1_ref, wd1_ref, rel8_ref, t1p_ref, u1_ref, c1p_ref):
    N = x_ref.shape[1]
    x = x_ref[0]
    x8 = jnp.concatenate([x, jnp.zeros((N, 5), jnp.float32)], axis=1)
    D = _pairdist(x)
    iota = jax.lax.broadcasted_iota(jnp.int32, (N, N), 1)
    Wd1 = wd1_ref[...]
    W1 = wsa1_ref[...]

    def round_common(r, D):
        sel, selF = _minsel(D, iota)
        nb8 = jnp.dot(selF, x8, preferred_element_type=jnp.float32,
                      precision=_PH)
        rel8 = nb8 - x8
        rel8_ref[0, r] = jnp.transpose(rel8)
        t1r = jnp.dot(rel8[:, 0:3], W1, preferred_element_type=jnp.float32)
        return jnp.where(sel, jnp.inf, D), rel8, t1r

    D, rel8_0, t1r0 = round_common(0, D)
    h0 = jnp.dot(jnp.concatenate([rel8_0[:, 0:3], x], axis=1), Wd1,
                 preferred_element_type=jnp.float32)
    c = jnp.sum(h0, axis=0, keepdims=True) / N
    u0 = h0 - c

    def body_a(r, state):
        D, maxH, acch, acc2h, at1, at2 = state
        D, rel8, t1r = round_common(r, D)
        h = jnp.dot(jnp.concatenate([rel8[:, 0:3], x], axis=1), Wd1,
                    preferred_element_type=jnp.float32)
        u = h - c
        return (D, jnp.maximum(maxH, h), acch + h, acc2h + u * u,
                at1 + t1r, at2 + t1r * t1r)

    def body_b(r, state):
        D, at1, at2 = state
        D, _, t1r = round_common(r, D)
        return (D, at1 + t1r, at2 + t1r * t1r)

    state = (D, h0, h0, u0 * u0, t1r0, t1r0 * t1r0)
    D, maxH, acch, acc2h, at1, at2 = jax.lax.fori_loop(1, 20, body_a, state)
    _, at1, at2 = jax.lax.fori_loop(20, 32, body_b, (D, at1, at2))

    u1_ref[0] = maxH
    t1p_ref[0, 0:1, :] = jnp.sum(at1, axis=0, keepdims=True)
    t1p_ref[0, 1:2, :] = jnp.sum(at2, axis=0, keepdims=True)
    c1p_ref[0, 0:1, :] = jnp.sum(acch, axis=0, keepdims=True)
    c1p_ref[0, 1:2, :] = jnp.sum(acc2h, axis=0, keepdims=True)
    c1p_ref[0, 2:3, :] = c


def _jacobi_rot(app, aqq, apq, arp, arq):
    absq = jnp.abs(apq)
    theta = (aqq - app) * 0.5 / jnp.where(absq > 0, apq, 1.0)
    sgn = jnp.where(theta >= 0, 1.0, -1.0)
    t = sgn / (jnp.abs(theta) + jnp.sqrt(theta * theta + 1.0))
    t = jnp.where(absq > 0, t, 0.0)
    c = jax.lax.rsqrt(t * t + 1.0)
    s = t * c
    app2 = app - t * apq
    aqq2 = aqq + t * apq
    arp2 = c * arp - s * arq
    arq2 = s * arp + c * arq
    return app2, aqq2, jnp.zeros_like(apq), arp2, arq2


def _eig_body(x_ref, we1_ref, be1_ref, we2_ref, be2_ref, h3t_ref):
    N = x_ref.shape[1]
    x = x_ref[0]
    d2 = jnp.zeros((N, N), jnp.float32)
    for c in range(3):
        col = x[:, c:c + 1]
        diff = col - jnp.reshape(col, (1, N))
        d2 = d2 + diff * diff

    eye = (jax.lax.broadcasted_iota(jnp.int32, (N, N), 0)
           == jax.lax.broadcasted_iota(jnp.int32, (N, N), 1))
    d = jnp.sqrt(d2 + 1e-12)
    mneg = jnp.max(jnp.where(eye, -jnp.inf, d), axis=1, keepdims=True)
    maxd = jnp.max(mneg, axis=0, keepdims=True)
    radius = maxd * 0.1
    dinf = jnp.where(eye, jnp.inf, d)
    maskF = (dinf < radius).astype(jnp.float32)

    xx = x * x
    xy = x[:, 0:1] * x[:, 1:2]
    xz = x[:, 0:1] * x[:, 2:3]
    yz = x[:, 1:2] * x[:, 2:3]
    F = jnp.concatenate([x, xx, xy, xz, yz], axis=1)
    ST = jax.lax.dot_general(F, maskF, (((0,), (1,)), ((), ())),
                             preferred_element_type=jnp.float32)
    cnt = jnp.sum(maskF, axis=0, keepdims=True)
    cntc = jnp.maximum(cnt, 1.0)
    mx = ST[0:1] / cntc
    my = ST[1:2] / cntc
    mz = ST[2:3] / cntc
    invn = 1.0 / N
    a11 = (ST[3:4] - cnt * mx * mx) * invn + 1e-6
    a22 = (ST[4:5] - cnt * my * my) * invn + 2e-6
    a33 = (ST[5:6] - cnt * mz * mz) * invn + 3e-6
    a12 = (ST[6:7] - cnt * mx * my) * invn
    a13 = (ST[7:8] - cnt * mx * mz) * invn
    a23 = (ST[8:9] - cnt * my * mz) * invn

    for _ in range(6):
        a11, a22, a12, a13, a23 = _jacobi_rot(a11, a22, a12, a13, a23)
        a11, a33, a13, a12, a23 = _jacobi_rot(a11, a33, a13, a12, a23)
        a22, a33, a23, a12, a13 = _jacobi_rot(a22, a33, a23, a12, a13)

    lo = jnp.minimum(jnp.minimum(a11, a22), a33)
    hi = jnp.maximum(jnp.maximum(a11, a22), a33)
    mid = (a11 + a22 + a33) - lo - hi
    evT = jnp.concatenate([lo, mid, hi], axis=0)

    t = jax.lax.dot_general(we1_ref[...], evT, (((0,), (0,)), ((), ())),
                            preferred_element_type=jnp.float32)
    t = jnp.maximum(t + be1_ref[...], 0.0)
    h3t = jax.lax.dot_general(we2_ref[...], t, (((0,), (0,)), ((), ())),
                              preferred_element_type=jnp.float32) + be2_ref[...]
    h3t_ref[0] = h3t


def _sa_t2_slices(rel8_ref, t1p_ref, wsa1_ref, wsa2_ref, B, K):
    N = rel8_ref.shape[3]
    mean1, den1 = _fin(t1p_ref, float(B * N * K))
    W1 = wsa1_ref[...]
    W2 = wsa2_ref[...]
    for r in range(K):
        rel = jnp.transpose(rel8_ref[0, r])
        t1 = jnp.dot(rel[:, 0:3], W1, preferred_element_type=jnp.float32)
        h = jnp.maximum((t1 - mean1) / den1, 0.0)
        yield jnp.dot(h, W2, preferred_element_type=jnp.float32)


def _sa_mid_body(rel8_ref, t1p_ref, wsa1_ref, wsa2_ref, t2p_ref, *, B, K):
    N = rel8_ref.shape[3]
    acc = acc2 = jnp.zeros((N, 128), jnp.float32)
    for t2 in _sa_t2_slices(rel8_ref, t1p_ref, wsa1_ref, wsa2_ref, B, K):
        acc = acc + t2
        acc2 = acc2 + t2 * t2
    t2p_ref[0, 0:1, :] = jnp.sum(acc, axis=0, keepdims=True)
    t2p_ref[0, 1:2, :] = jnp.sum(acc2, axis=0, keepdims=True)


def _sa_out_body(rel8_ref, t1p_ref, wsa1_ref, wsa2_ref, t2p_ref, h1_ref, *, B, K):
    N = rel8_ref.shape[3]
    mean2, den2 = _fin(t2p_ref, float(B * N * K))
    acc = jnp.full((N, 128), -jnp.inf, jnp.float32)
    for t2 in _sa_t2_slices(rel8_ref, t1p_ref, wsa1_ref, wsa2_ref, B, K):
        acc = jnp.maximum(acc, jnp.maximum((t2 - mean2) / den2, 0.0))
    h1_ref[0] = acc


def _conv_body(up_ref, pp_ref, w_ref, x_ref, u_ref, p_ref, *, B, K, din):
    N = up_ref.shape[1]
    mean, den = _fin2(pp_ref, float(B * N * K))
    X = _leaky((up_ref[0] - mean) / den)
    x_ref[0] = X
    D = _pairdist(X)
    iota = jax.lax.broadcasted_iota(jnp.int32, (N, N), 1)
    W = w_ref[...]

    def one_round(D):
        sel, selF = _minsel(D, iota)
        nb = jnp.dot(selF, X, preferred_element_type=jnp.float32,
                     precision=_PH)
        feat = jnp.concatenate([nb - X, X], axis=1)
        h = jnp.dot(feat, W, preferred_element_type=jnp.float32)
        return jnp.where(sel, jnp.inf, D), h

    D, h0 = one_round(D)
    c = jnp.sum(h0, axis=0, keepdims=True) / N
    u0 = h0 - c
    state = (D, h0, h0, u0 * u0)

    def body(r, state):
        D, maxH, acc, acc2 = state
        D, h = one_round(D)
        u = h - c
        return (D, jnp.maximum(maxH, h), acc + h, acc2 + u * u)

    _, maxH, acc, acc2 = jax.lax.fori_loop(1, K, body, state)
    u_ref[0] = maxH
    p_ref[0, 0:1, :] = jnp.sum(acc, axis=0, keepdims=True)
    p_ref[0, 1:2, :] = jnp.sum(acc2, axis=0, keepdims=True)
    p_ref[0, 2:3, :] = c


def _mlp_body(t_ref, pp_ref, w_ref, o_ref, p_ref, *, B, K, act):
    N = t_ref.shape[1]
    mean, den = _fin(pp_ref, float(B * N * K))
    h = (t_ref[0] - mean) / den
    h = _leaky(h) if act == "leaky" else jnp.maximum(h, 0.0)
    t2 = jnp.dot(h, w_ref[...], preferred_element_type=jnp.float32)
    o_ref[0] = t2
    s, ss = _rowsums(t2)
    p_ref[0, 0:1, :] = s
    p_ref[0, 1:2, :] = ss


def _d1_body(x1_ref, x2_ref, x3_ref, u4_ref, p4_ref, w_ref, o_ref, p_ref, *, B, K):
    mean, den = _fin2(p4_ref, float(B * u4_ref.shape[1] * K))
    x4 = _leaky((u4_ref[0] - mean) / den)
    xc = jnp.concatenate([x1_ref[0], x2_ref[0], x3_ref[0], x4], axis=1)
    t = jnp.dot(xc, w_ref[...], preferred_element_type=jnp.float32)
    o_ref[0] = t
    s, ss = _rowsums(t)
    p_ref[0, 0:1, :] = s
    p_ref[0, 1:2, :] = ss


def _d5_body(h1_ref, tg3_ref, pg3_ref, h3_ref, w_ref, o_ref, p_ref, *, B):
    N = h1_ref.shape[1]
    mean, den = _fin(pg3_ref, float(B * N))
    h2 = jnp.maximum((tg3_ref[0] - mean) / den, 0.0)
    z = jnp.concatenate([h1_ref[0], h2, h3_ref[0]], axis=1)
    t = jnp.dot(z, w_ref[...], preferred_element_type=jnp.float32)
    o_ref[0] = t
    s, ss = _rowsums(t)
    p_ref[0, 0:1, :] = s
    p_ref[0, 1:2, :] = ss


def _final_body(t_ref, pp_ref, z_ref, *, B):
    N = t_ref.shape[1]
    mean, den = _fin(pp_ref, float(B * N))
    z_ref[0] = jnp.maximum((t_ref[0] - mean) / den, 0.0)


def _full(shape, dtype=jnp.float32):
    return jax.ShapeDtypeStruct(shape, dtype)


def _spec_b(*blk):
    nd = len(blk)
    return pl.BlockSpec((1,) + blk, lambda b: (b,) + (0,) * nd)


def _spec_w(shape):
    nd = len(shape)
    return pl.BlockSpec(shape, lambda b: (0,) * nd)


def kernel(pointcloud, W_sa1, W_sa2, Wd1, Wd2, Wd3, Wd4, Wd5, Wg1, Wg2, Wg3,
           We1, be1, We2, be2, Wc1, Wc2, Wc3, numpoints):
    B, N, _ = pointcloud.shape
    f32 = jnp.float32
    xyz = pointcloud[..., 0:3]

    def call(body, ins, in_specs, out_shapes, out_specs, **kw):
        return pl.pallas_call(
            functools.partial(body, **kw),
            grid=(B,),
            in_specs=in_specs,
            out_specs=out_specs,
            out_shape=out_shapes,
        )(*ins)

    rel8, t1p, U1, c1p = call(
        _geo_body,
        (xyz, W_sa1, Wd1),
        [_spec_b(N, 3), _spec_w((3, 64)), _spec_w((6, 64))],
        (_full((B, 32, 8, N)), _full((B, 2, 64)), _full((B, N, 64)),
         _full((B, 3, 64))),
        (_spec_b(32, 8, N), _spec_b(2, 64), _spec_b(N, 64), _spec_b(3, 64)),
    )

    be1c, be2c = be1[:, None], be2[:, None]
    h3t = call(
        _eig_body,
        (xyz, We1, be1c, We2, be2c),
        [_spec_b(N, 3), _spec_w((3, 4)), _spec_w((4, 1)), _spec_w((4, 4)),
         _spec_w((4, 1))],
        _full((B, 4, N)),
        _spec_b(4, N),
    )
    h3 = jnp.transpose(h3t, (0, 2, 1))

    t2p = call(
        _sa_mid_body,
        (rel8, t1p, W_sa1, W_sa2),
        [_spec_b(32, 8, N), _spec_w((B, 2, 64)), _spec_w((3, 64)),
         _spec_w((64, 128))],
        _full((B, 2, 128)),
        _spec_b(2, 128),
        B=B, K=32,
    )
    h1 = call(
        _sa_out_body,
        (rel8, t1p, W_sa1, W_sa2, t2p),
        [_spec_b(32, 8, N), _spec_w((B, 2, 64)), _spec_w((3, 64)),
         _spec_w((64, 128)), _spec_w((B, 2, 128))],
        _full((B, N, 128)),
        _spec_b(N, 128),
        B=B, K=32,
    )

    def conv(U, pp, W, din, dout):
        return call(
            _conv_body,
            (U, pp, W),
            [_spec_b(N, din), _spec_w((B, 3, din)), _spec_w((2 * din, dout))],
            (_full((B, N, din)), _full((B, N, dout)), _full((B, 3, dout))),
            (_spec_b(N, din), _spec_b(N, dout), _spec_b(3, dout)),
            B=B, K=20, din=din,
        )

    x1, U2, c2p = conv(U1, c1p, Wd2, 64, 64)
    x2, U3, c3p = conv(U2, c2p, Wd3, 64, 128)
    x3, U4, c4p = conv(U3, c3p, Wd4, 128, 256)

    t5, p5 = call(
        _d1_body,
        (x1, x2, x3, U4, c4p, Wd5),
        [_spec_b(N, 64), _spec_b(N, 64), _spec_b(N, 128), _spec_b(N, 256),
         _spec_w((B, 3, 256)), _spec_w((512, 1024))],
        (_full((B, N, 1024)), _full((B, 2, 1024))),
        (_spec_b(N, 1024), _spec_b(2, 1024)),
        B=B, K=20,
    )

    def mlp(T, pp, W, cin, cout, act, K=1):
        return call(
            _mlp_body,
            (T, pp, W),
            [_spec_b(N, cin), _spec_w((B, 2, cin)), _spec_w((cin, cout))],
            (_full((B, N, cout)), _full((B, 2, cout))),
            (_spec_b(N, cout), _spec_b(2, cout)),
            B=B, K=K, act=act,
        )

    tg1, pg1 = mlp(t5, p5, Wg1, 1024, 256, "leaky")
    tg2, pg2 = mlp(tg1, pg1, Wg2, 256, 64, "relu")
    tg3, pg3 = mlp(tg2, pg2, Wg3, 64, 32, "relu")

    tc1, pc1 = call(
        _d5_body,
        (h1, tg3, pg3, h3, Wc1),
        [_spec_b(N, 128), _spec_b(N, 32), _spec_w((B, 2, 32)), _spec_b(N, 4),
         _spec_w((164, 512))],
        (_full((B, N, 512)), _full((B, 2, 512))),
        (_spec_b(N, 512), _spec_b(2, 512)),
        B=B,
    )
    tc2, pc2 = mlp(tc1, pc1, Wc2, 512, 256, "relu")
    tc3, pc3 = mlp(tc2, pc2, Wc3, 256, 128, "relu")

    z = call(
        _final_body,
        (tc3, pc3),
        [_spec_b(N, 128), _spec_w((B, 2, 128))],
        _full((B, N, 128)),
        _spec_b(N, 128),
        B=B,
    )
    return xyz, jnp.transpose(z, (0, 2, 1))


# speedup vs baseline: 1.0332x; 1.0332x over previous
import functools

import jax
import jax.numpy as jnp
from jax.experimental import pallas as pl

_BNEPS = 1e-5
_LEAK = 0.2


_PH = jax.lax.Precision.HIGHEST


def _fin(pp_ref, count):
    s = jnp.sum(pp_ref[:, 0, :], axis=0)
    ss = jnp.sum(pp_ref[:, 1, :], axis=0)
    mean = s / count
    var = ss / count - mean * mean
    den = jnp.sqrt(var + _BNEPS)
    return mean, den


def _fin2(pp_ref, count):
    B = pp_ref.shape[0]
    nb = count / B
    m = jnp.sum(pp_ref[:, 0, :], axis=0) / count
    dev = jnp.zeros_like(m)
    for b in range(B):
        s_b = pp_ref[b, 0, :]
        ss2_b = pp_ref[b, 1, :]
        c_b = pp_ref[b, 2, :]
        d = m - c_b
        dev = dev + (ss2_b - 2.0 * d * (s_b - nb * m) - nb * d * d)
    var = dev / count
    den = jnp.sqrt(var + _BNEPS)
    return m, den


def _leaky(x):
    return jnp.where(x >= 0, x, _LEAK * x)


def _rowsums(t):
    return jnp.sum(t, axis=0, keepdims=True), jnp.sum(t * t, axis=0, keepdims=True)


def _pairdist(X):
    N = X.shape[0]
    sq = jnp.sum(X * X, axis=1, keepdims=True)
    G = jax.lax.dot_general(X, X, (((1,), (1,)), ((), ())),
                            preferred_element_type=jnp.float32)
    return (sq - 2.0 * G) + jnp.reshape(sq, (1, N))


def _minsel(D, iota):
    N = D.shape[0]
    m = jnp.min(D, axis=1, keepdims=True)
    idx = jnp.min(jnp.where(D == m, iota, N), axis=1, keepdims=True)
    sel = iota == idx
    return sel, sel.astype(jnp.float32)


def _geo_body(x_ref, wsa1_ref, wd1_ref, rel8_ref, t1p_ref, u1_ref, c1p_ref):
    N = x_ref.shape[1]
    x = x_ref[0]
    x8 = jnp.concatenate([x, jnp.zeros((N, 5), jnp.float32)], axis=1)
    D = _pairdist(x)
    iota = jax.lax.broadcasted_iota(jnp.int32, (N, N), 1)
    Wd1 = wd1_ref[...]
    W1 = wsa1_ref[...]

    def round_common(r, D):
        sel, selF = _minsel(D, iota)
        nb8 = jnp.dot(selF, x8, preferred_element_type=jnp.float32,
                      precision=_PH)
        rel8 = nb8 - x8
        rel8_ref[0, r] = jnp.transpose(rel8)
        t1r = jnp.dot(rel8[:, 0:3], W1, preferred_element_type=jnp.float32)
        s, ss = _rowsums(t1r)
        return jnp.where(sel, jnp.inf, D), rel8, s, ss

    D, rel8_0, st, sst = round_common(0, D)
    h0 = jnp.dot(jnp.concatenate([rel8_0[:, 0:3], x], axis=1), Wd1,
                 preferred_element_type=jnp.float32)
    c = jnp.sum(h0, axis=0, keepdims=True) / N
    u0 = h0 - c
    sc = jnp.sum(h0, axis=0, keepdims=True)
    ssc = jnp.sum(u0 * u0, axis=0, keepdims=True)

    def body_a(r, state):
        D, maxH, sc, ssc, st, sst = state
        D, rel8, s, ss = round_common(r, D)
        h = jnp.dot(jnp.concatenate([rel8[:, 0:3], x], axis=1), Wd1,
                    preferred_element_type=jnp.float32)
        u = h - c
        return (D, jnp.maximum(maxH, h), sc + jnp.sum(h, axis=0, keepdims=True),
                ssc + jnp.sum(u * u, axis=0, keepdims=True), st + s, sst + ss)

    def body_b(r, state):
        D, st, sst = state
        D, _, s, ss = round_common(r, D)
        return (D, st + s, sst + ss)

    state = (D, h0, sc, ssc, st, sst)
    D, maxH, sc, ssc, st, sst = jax.lax.fori_loop(1, 20, body_a, state)
    _, st, sst = jax.lax.fori_loop(20, 32, body_b, (D, st, sst))

    u1_ref[0] = maxH
    t1p_ref[0, 0:1, :] = st
    t1p_ref[0, 1:2, :] = sst
    c1p_ref[0, 0:1, :] = sc
    c1p_ref[0, 1:2, :] = ssc
    c1p_ref[0, 2:3, :] = c


def _jacobi_rot(app, aqq, apq, arp, arq):
    absq = jnp.abs(apq)
    theta = (aqq - app) * 0.5 / jnp.where(absq > 0, apq, 1.0)
    sgn = jnp.where(theta >= 0, 1.0, -1.0)
    t = sgn / (jnp.abs(theta) + jnp.sqrt(theta * theta + 1.0))
    t = jnp.where(absq > 0, t, 0.0)
    c = jax.lax.rsqrt(t * t + 1.0)
    s = t * c
    app2 = app - t * apq
    aqq2 = aqq + t * apq
    arp2 = c * arp - s * arq
    arq2 = s * arp + c * arq
    return app2, aqq2, jnp.zeros_like(apq), arp2, arq2


def _eig_body(x_ref, we1_ref, be1_ref, we2_ref, be2_ref, h3t_ref):
    N = x_ref.shape[1]
    x = x_ref[0]
    d2 = jnp.zeros((N, N), jnp.float32)
    for c in range(3):
        col = x[:, c:c + 1]
        diff = col - jnp.reshape(col, (1, N))
        d2 = d2 + diff * diff

    eye = (jax.lax.broadcasted_iota(jnp.int32, (N, N), 0)
           == jax.lax.broadcasted_iota(jnp.int32, (N, N), 1))
    d = jnp.sqrt(d2 + 1e-12)
    mneg = jnp.max(jnp.where(eye, -jnp.inf, d), axis=1, keepdims=True)
    maxd = jnp.max(mneg, axis=0, keepdims=True)
    radius = maxd * 0.1
    dinf = jnp.where(eye, jnp.inf, d)
    maskF = (dinf < radius).astype(jnp.float32)

    xx = x * x
    xy = x[:, 0:1] * x[:, 1:2]
    xz = x[:, 0:1] * x[:, 2:3]
    yz = x[:, 1:2] * x[:, 2:3]
    F = jnp.concatenate([x, xx, xy, xz, yz], axis=1)
    ST = jax.lax.dot_general(F, maskF, (((0,), (1,)), ((), ())),
                             preferred_element_type=jnp.float32)
    cnt = jnp.sum(maskF, axis=0, keepdims=True)
    cntc = jnp.maximum(cnt, 1.0)
    mx = ST[0:1] / cntc
    my = ST[1:2] / cntc
    mz = ST[2:3] / cntc
    invn = 1.0 / N
    a11 = (ST[3:4] - cnt * mx * mx) * invn + 1e-6
    a22 = (ST[4:5] - cnt * my * my) * invn + 2e-6
    a33 = (ST[5:6] - cnt * mz * mz) * invn + 3e-6
    a12 = (ST[6:7] - cnt * mx * my) * invn
    a13 = (ST[7:8] - cnt * mx * mz) * invn
    a23 = (ST[8:9] - cnt * my * mz) * invn

    for _ in range(6):
        a11, a22, a12, a13, a23 = _jacobi_rot(a11, a22, a12, a13, a23)
        a11, a33, a13, a12, a23 = _jacobi_rot(a11, a33, a13, a12, a23)
        a22, a33, a23, a12, a13 = _jacobi_rot(a22, a33, a23, a12, a13)

    lo = jnp.minimum(jnp.minimum(a11, a22), a33)
    hi = jnp.maximum(jnp.maximum(a11, a22), a33)
    mid = (a11 + a22 + a33) - lo - hi
    evT = jnp.concatenate([lo, mid, hi], axis=0)

    t = jax.lax.dot_general(we1_ref[...], evT, (((0,), (0,)), ((), ())),
                            preferred_element_type=jnp.float32)
    t = jnp.maximum(t + be1_ref[...], 0.0)
    h3t = jax.lax.dot_general(we2_ref[...], t, (((0,), (0,)), ((), ())),
                              preferred_element_type=jnp.float32) + be2_ref[...]
    h3t_ref[0] = h3t


def _sa_t2_slices(rel8_ref, t1p_ref, wsa1_ref, wsa2_ref, B, K):
    N = rel8_ref.shape[3]
    mean1, den1 = _fin(t1p_ref, float(B * N * K))
    W1 = wsa1_ref[...]
    W2 = wsa2_ref[...]
    for r in range(K):
        rel = jnp.transpose(rel8_ref[0, r])
        t1 = jnp.dot(rel[:, 0:3], W1, preferred_element_type=jnp.float32)
        h = jnp.maximum((t1 - mean1) / den1, 0.0)
        yield jnp.dot(h, W2, preferred_element_type=jnp.float32)


def _sa_mid_body(rel8_ref, t1p_ref, wsa1_ref, wsa2_ref, t2p_ref, *, B, K):
    s = ss = jnp.zeros((1, 128), jnp.float32)
    for t2 in _sa_t2_slices(rel8_ref, t1p_ref, wsa1_ref, wsa2_ref, B, K):
        sr, ssr = _rowsums(t2)
        s, ss = s + sr, ss + ssr
    t2p_ref[0, 0:1, :] = s
    t2p_ref[0, 1:2, :] = ss


def _sa_out_body(rel8_ref, t1p_ref, wsa1_ref, wsa2_ref, t2p_ref, h1_ref, *, B, K):
    N = rel8_ref.shape[3]
    mean2, den2 = _fin(t2p_ref, float(B * N * K))
    acc = jnp.full((N, 128), -jnp.inf, jnp.float32)
    for t2 in _sa_t2_slices(rel8_ref, t1p_ref, wsa1_ref, wsa2_ref, B, K):
        acc = jnp.maximum(acc, jnp.maximum((t2 - mean2) / den2, 0.0))
    h1_ref[0] = acc


def _conv_body(up_ref, pp_ref, w_ref, x_ref, u_ref, p_ref, *, B, K, din):
    N = up_ref.shape[1]
    mean, den = _fin2(pp_ref, float(B * N * K))
    X = _leaky((up_ref[0] - mean) / den)
    x_ref[0] = X
    D = _pairdist(X)
    iota = jax.lax.broadcasted_iota(jnp.int32, (N, N), 1)
    W = w_ref[...]

    def one_round(D):
        sel, selF = _minsel(D, iota)
        nb = jnp.dot(selF, X, preferred_element_type=jnp.float32,
                     precision=_PH)
        feat = jnp.concatenate([nb - X, X], axis=1)
        h = jnp.dot(feat, W, preferred_element_type=jnp.float32)
        return jnp.where(sel, jnp.inf, D), h

    D, h0 = one_round(D)
    c = jnp.sum(h0, axis=0, keepdims=True) / N
    u0 = h0 - c
    state = (D, h0, jnp.sum(h0, axis=0, keepdims=True),
             jnp.sum(u0 * u0, axis=0, keepdims=True))

    def body(r, state):
        D, maxH, s, ss2 = state
        D, h = one_round(D)
        u = h - c
        return (D, jnp.maximum(maxH, h), s + jnp.sum(h, axis=0, keepdims=True),
                ss2 + jnp.sum(u * u, axis=0, keepdims=True))

    _, maxH, s, ss2 = jax.lax.fori_loop(1, K, body, state)
    u_ref[0] = maxH
    p_ref[0, 0:1, :] = s
    p_ref[0, 1:2, :] = ss2
    p_ref[0, 2:3, :] = c


def _mlp_body(t_ref, pp_ref, w_ref, o_ref, p_ref, *, B, K, act):
    N = t_ref.shape[1]
    mean, den = _fin(pp_ref, float(B * N * K))
    h = (t_ref[0] - mean) / den
    h = _leaky(h) if act == "leaky" else jnp.maximum(h, 0.0)
    t2 = jnp.dot(h, w_ref[...], preferred_element_type=jnp.float32)
    o_ref[0] = t2
    s, ss = _rowsums(t2)
    p_ref[0, 0:1, :] = s
    p_ref[0, 1:2, :] = ss


def _d1_body(x1_ref, x2_ref, x3_ref, u4_ref, p4_ref, w_ref, o_ref, p_ref, *, B, K):
    mean, den = _fin2(p4_ref, float(B * u4_ref.shape[1] * K))
    x4 = _leaky((u4_ref[0] - mean) / den)
    xc = jnp.concatenate([x1_ref[0], x2_ref[0], x3_ref[0], x4], axis=1)
    t = jnp.dot(xc, w_ref[...], preferred_element_type=jnp.float32)
    o_ref[0] = t
    s, ss = _rowsums(t)
    p_ref[0, 0:1, :] = s
    p_ref[0, 1:2, :] = ss


def _d5_body(h1_ref, tg3_ref, pg3_ref, h3_ref, w_ref, o_ref, p_ref, *, B):
    N = h1_ref.shape[1]
    mean, den = _fin(pg3_ref, float(B * N))
    h2 = jnp.maximum((tg3_ref[0] - mean) / den, 0.0)
    z = jnp.concatenate([h1_ref[0], h2, h3_ref[0]], axis=1)
    t = jnp.dot(z, w_ref[...], preferred_element_type=jnp.float32)
    o_ref[0] = t
    s, ss = _rowsums(t)
    p_ref[0, 0:1, :] = s
    p_ref[0, 1:2, :] = ss


def _final_body(t_ref, pp_ref, z_ref, *, B):
    N = t_ref.shape[1]
    mean, den = _fin(pp_ref, float(B * N))
    z_ref[0] = jnp.maximum((t_ref[0] - mean) / den, 0.0)


def _full(shape, dtype=jnp.float32):
    return jax.ShapeDtypeStruct(shape, dtype)


def _spec_b(*blk):
    nd = len(blk)
    return pl.BlockSpec((1,) + blk, lambda b: (b,) + (0,) * nd)


def _spec_w(shape):
    nd = len(shape)
    return pl.BlockSpec(shape, lambda b: (0,) * nd)


def kernel(pointcloud, W_sa1, W_sa2, Wd1, Wd2, Wd3, Wd4, Wd5, Wg1, Wg2, Wg3,
           We1, be1, We2, be2, Wc1, Wc2, Wc3, numpoints):
    B, N, _ = pointcloud.shape
    f32 = jnp.float32
    xyz = pointcloud[..., 0:3]

    def call(body, ins, in_specs, out_shapes, out_specs, **kw):
        return pl.pallas_call(
            functools.partial(body, **kw),
            grid=(B,),
            in_specs=in_specs,
            out_specs=out_specs,
            out_shape=out_shapes,
        )(*ins)

    rel8, t1p, U1, c1p = call(
        _geo_body,
        (xyz, W_sa1, Wd1),
        [_spec_b(N, 3), _spec_w((3, 64)), _spec_w((6, 64))],
        (_full((B, 32, 8, N)), _full((B, 2, 64)), _full((B, N, 64)),
         _full((B, 3, 64))),
        (_spec_b(32, 8, N), _spec_b(2, 64), _spec_b(N, 64), _spec_b(3, 64)),
    )

    be1c, be2c = be1[:, None], be2[:, None]
    h3t = call(
        _eig_body,
        (xyz, We1, be1c, We2, be2c),
        [_spec_b(N, 3), _spec_w((3, 4)), _spec_w((4, 1)), _spec_w((4, 4)),
         _spec_w((4, 1))],
        _full((B, 4, N)),
        _spec_b(4, N),
    )
    h3 = jnp.transpose(h3t, (0, 2, 1))

    t2p = call(
        _sa_mid_body,
        (rel8, t1p, W_sa1, W_sa2),
        [_spec_b(32, 8, N), _spec_w((B, 2, 64)), _spec_w((3, 64)),
         _spec_w((64, 128))],
        _full((B, 2, 128)),
        _spec_b(2, 128),
        B=B, K=32,
    )
    h1 = call(
        _sa_out_body,
        (rel8, t1p, W_sa1, W_sa2, t2p),
        [_spec_b(32, 8, N), _spec_w((B, 2, 64)), _spec_w((3, 64)),
         _spec_w((64, 128)), _spec_w((B, 2, 128))],
        _full((B, N, 128)),
        _spec_b(N, 128),
        B=B, K=32,
    )

    def conv(U, pp, W, din, dout):
        return call(
            _conv_body,
            (U, pp, W),
            [_spec_b(N, din), _spec_w((B, 3, din)), _spec_w((2 * din, dout))],
            (_full((B, N, din)), _full((B, N, dout)), _full((B, 3, dout))),
            (_spec_b(N, din), _spec_b(N, dout), _spec_b(3, dout)),
            B=B, K=20, din=din,
        )

    x1, U2, c2p = conv(U1, c1p, Wd2, 64, 64)
    x2, U3, c3p = conv(U2, c2p, Wd3, 64, 128)
    x3, U4, c4p = conv(U3, c3p, Wd4, 128, 256)

    t5, p5 = call(
        _d1_body,
        (x1, x2, x3, U4, c4p, Wd5),
        [_spec_b(N, 64), _spec_b(N, 64), _spec_b(N, 128), _spec_b(N, 256),
         _spec_w((B, 3, 256)), _spec_w((512, 1024))],
        (_full((B, N, 1024)), _full((B, 2, 1024))),
        (_spec_b(N, 1024), _spec_b(2, 1024)),
        B=B, K=20,
    )

    def mlp(T, pp, W, cin, cout, act, K=1):
        return call(
            _mlp_body,
            (T, pp, W),
            [_spec_b(N, cin), _spec_w((B, 2, cin)), _spec_w((cin, cout))],
            (_full((B, N, cout)), _full((B, 2, cout))),
            (_spec_b(N, cout), _spec_b(2, cout)),
            B=B, K=K, act=act,
        )

    tg1, pg1 = mlp(t5, p5, Wg1, 1024, 256, "leaky")
    tg2, pg2 = mlp(tg1, pg1, Wg2, 256, 64, "relu")
    tg3, pg3 = mlp(tg2, pg2, Wg3, 64, 32, "relu")

    tc1, pc1 = call(
        _d5_body,
        (h1, tg3, pg3, h3, Wc1),
        [_spec_b(N, 128), _spec_b(N, 32), _spec_w((B, 2, 32)), _spec_b(N, 4),
         _spec_w((164, 512))],
        (_full((B, N, 512)), _full((B, 2, 512))),
        (_spec_b(N, 512), _spec_b(2, 512)),
        B=B,
    )
    tc2, pc2 = mlp(tc1, pc1, Wc2, 512, 256, "relu")
    tc3, pc3 = mlp(tc2, pc2, Wc3, 256, 128, "relu")

    z = call(
        _final_body,
        (tc3, pc3),
        [_spec_b(N, 128), _spec_w((B, 2, 128))],
        _full((B, N, 128)),
        _spec_b(N, 128),
        B=B,
    )
    return xyz, jnp.transpose(z, (0, 2, 1))
